# baseline (speedup 1.0000x reference)
import jax
import jax.numpy as jnp
from jax import lax
from jax.experimental import pallas as pl
from jax.experimental.pallas import tpu as pltpu
from jax.experimental.pallas import tpu_sc as plsc

VOCAB_PAD = 1024
EMB = 64
SCALE = 8.0
NC = 2
NS = 16
NW = NC * NS
B_TOTAL = 4096 * 200
B_PER_W = B_TOTAL // NW
CHUNK = 400
N_CHUNKS = B_PER_W // CHUNK
PS = 192
PT = CHUNK - PS
T_GROUPS = PT // 16
ROWS_PER_TILE = VOCAB_PAD // NS
TAB_WORDS = VOCAB_PAD * EMB

_GATHER_DNUMS = lax.GatherDimensionNumbers(
    offset_dims=(), collapsed_slice_dims=(0,), start_index_map=(0,))


def _body(x_hbm, tab_hbm, tabflat_hbm, out_hbm, shared, tabv, tbuf,
          idx0, idx1, rows0, rows1, gsem0, gsem1, osem0, osem1, isem0, isem1):
    s = lax.axis_index("s")
    wid = s * NC + lax.axis_index("c")
    rows = (rows0, rows1)
    idxb = (idx0, idx1)
    gsem = (gsem0, gsem1)
    osem = (osem0, osem1)
    isem = (isem0, isem1)
    base = wid * B_PER_W

    pltpu.sync_copy(tab_hbm.at[pl.ds(s * ROWS_PER_TILE, ROWS_PER_TILE)], tbuf)

    def scale_row(r, carry):
        for j in range(EMB // 16):
            tbuf[r, pl.ds(j * 16, 16)] = tbuf[r, pl.ds(j * 16, 16)] * SCALE
        return carry

    lax.fori_loop(0, ROWS_PER_TILE, scale_row, 0)
    pltpu.sync_copy(tbuf, shared.at[pl.ds(s * ROWS_PER_TILE, ROWS_PER_TILE)])

    pltpu.sync_copy(tabflat_hbm.at[pl.ds(0, TAB_WORDS)], tabv)

    pltpu.sync_copy(x_hbm.at[pl.ds(base, CHUNK)], idx0)
    pltpu.async_copy(x_hbm.at[pl.ds(base + CHUNK, CHUNK)], idx1, isem1)
    plsc.subcore_barrier()

    def issue_gather(g, b):
        pltpu.async_copy(
            shared.at[idxb[b].at[pl.ds(0, PS)]], rows[b].at[pl.ds(0, PS)],
            gsem[b])

    def wait(sem, ref, dummy):
        pltpu.make_async_copy(dummy, ref, sem).wait()

    issue_gather(0, 0)
    cols = [lax.iota(jnp.int32, 16) + 16 * j for j in range(4)]

    def compute_part(b):
        @plsc.parallel_loop(0, T_GROUPS, unroll=1)
        def group(grp):
            vec = idxb[b][pl.ds(PS + grp * 16, 16)] * EMB

            for i in range(16):
                bc = lax.gather(
                    vec, jnp.full((16, 1), i, jnp.int32), _GATHER_DNUMS, (1,),
                    mode=lax.GatherScatterMode.PROMISE_IN_BOUNDS)
                r = PS + grp * 16 + i
                for j in range(4):
                    rows[b][r, pl.ds(16 * j, 16)] = plsc.load_gather(
                        tabv, [bc + cols[j]])

    def pair(gg, carry):
        for b in range(2):
            bp = 1 - b
            g = gg * 2 + b
            compute_part(b)
            wait(gsem[b], rows[b].at[pl.ds(0, PS)],
                 out_hbm.at[pl.ds(0, PS)])

            @pl.when(g + 2 < N_CHUNKS)
            def _():
                pltpu.async_copy(x_hbm.at[pl.ds(base + (g + 2) * CHUNK, CHUNK)],
                                 idxb[b], isem[b])

            @pl.when(g >= 1)
            def _():
                wait(osem[bp], rows[bp], out_hbm.at[pl.ds(0, CHUNK)])

            @pl.when(g + 1 < N_CHUNKS)
            def _():
                wait(isem[bp], idxb[bp], x_hbm.at[pl.ds(0, CHUNK)])
                issue_gather(g + 1, bp)

            pltpu.async_copy(rows[b],
                             out_hbm.at[pl.ds(base + g * CHUNK, CHUNK)],
                             osem[b])
        return carry

    lax.fori_loop(0, N_CHUNKS // 2, pair, 0)
    wait(osem[1], rows[1], out_hbm.at[pl.ds(0, CHUNK)])


_sc_call = pl.kernel(
    _body,
    out_type=jax.ShapeDtypeStruct((B_TOTAL, EMB), jnp.float32),
    mesh=plsc.VectorSubcoreMesh(
        core_axis_name="c", subcore_axis_name="s", num_cores=NC, num_subcores=NS
    ),
    scratch_types=[
        pltpu.VMEM_SHARED((VOCAB_PAD, EMB), jnp.float32),
        pltpu.VMEM((TAB_WORDS,), jnp.float32),
        pltpu.VMEM((ROWS_PER_TILE, EMB), jnp.float32),
        pltpu.VMEM((CHUNK,), jnp.int32),
        pltpu.VMEM((CHUNK,), jnp.int32),
        pltpu.VMEM((CHUNK, EMB), jnp.float32),
        pltpu.VMEM((CHUNK, EMB), jnp.float32),
        pltpu.SemaphoreType.DMA,
        pltpu.SemaphoreType.DMA,
        pltpu.SemaphoreType.DMA,
        pltpu.SemaphoreType.DMA,
        pltpu.SemaphoreType.DMA,
        pltpu.SemaphoreType.DMA,
    ],
    compiler_params=pltpu.CompilerParams(use_tc_tiling_on_sc=False,
                                         needs_layout_passes=False),
)


def kernel(x, table):
    tab = jnp.pad(table, ((0, VOCAB_PAD - table.shape[0]), (0, 0)))
    tabflat = tab.reshape(-1) * jnp.float32(SCALE)
    out = _sc_call(x.reshape(-1), tab, tabflat)
    return out.reshape(x.shape[0], x.shape[1], EMB)


# speedup vs baseline: 1.0085x; 1.0085x over previous
import jax
import jax.numpy as jnp
from jax import lax
from jax.experimental import pallas as pl
from jax.experimental.pallas import tpu as pltpu
from jax.experimental.pallas import tpu_sc as plsc

VOCAB_PAD = 1024
EMB = 64
SCALE = 8.0
NC = 2
NS = 16
NW = NC * NS
B_TOTAL = 4096 * 200
B_PER_W = B_TOTAL // NW
SUB = 128
SUBS = 4
CHUNK = 640
N_CHUNKS = B_PER_W // CHUNK
IDX_ROWS = B_PER_W // SUB
ROWS_PER_TILE = VOCAB_PAD // NS


def _body(x_hbm, tab_hbm, out_hbm, shared, idxbuf, rows0, rows1,
          gsem0, gsem1, osem0, osem1):
    s = lax.axis_index("s")
    wid = s * NC + lax.axis_index("c")
    rows = (rows0, rows1)
    gsem = (gsem0, gsem1)
    osem = (osem0, osem1)

    tstage = rows0.at[pl.ds(0, ROWS_PER_TILE)]
    pltpu.sync_copy(tab_hbm.at[pl.ds(s * ROWS_PER_TILE, ROWS_PER_TILE)], tstage)

    def scale_row(r, carry):
        for j in range(EMB // 16):
            rows0[r, pl.ds(j * 16, 16)] = rows0[r, pl.ds(j * 16, 16)] * SCALE
        return carry

    lax.fori_loop(0, ROWS_PER_TILE, scale_row, 0)
    pltpu.sync_copy(tstage, shared.at[pl.ds(s * ROWS_PER_TILE, ROWS_PER_TILE)])

    pltpu.sync_copy(x_hbm.at[pl.ds(wid * B_PER_W, B_PER_W)], idxbuf)
    plsc.subcore_barrier()

    def issue_gather(g, b):
        pltpu.async_copy(
            shared.at[idxbuf.at[pl.ds(g * CHUNK, CHUNK)]],
            rows[b], gsem[b])

    def wait_chunk(sem, b):
        pltpu.make_async_copy(out_hbm.at[0], rows[b], sem).wait()

    issue_gather(0, 0)
    cbase = wid * N_CHUNKS

    def pair(gg, carry):
        for b in range(2):
            bp = 1 - b
            g = gg * 2 + b
            wait_chunk(gsem[b], b)

            @pl.when(g + 1 < N_CHUNKS)
            def _():
                @pl.when(g >= 1)
                def _():
                    wait_chunk(osem[bp], bp)
                issue_gather(g + 1, bp)

            pltpu.async_copy(rows[b], out_hbm.at[cbase + g], osem[b])
        return carry

    lax.fori_loop(0, N_CHUNKS // 2, pair, 0)
    wait_chunk(osem[0], 0)
    wait_chunk(osem[1], 1)


_sc_call = pl.kernel(
    _body,
    out_type=jax.ShapeDtypeStruct((NW * N_CHUNKS, CHUNK, EMB), jnp.float32),
    mesh=plsc.VectorSubcoreMesh(
        core_axis_name="c", subcore_axis_name="s", num_cores=NC, num_subcores=NS
    ),
    scratch_types=[
        pltpu.VMEM_SHARED((VOCAB_PAD, EMB), jnp.float32),
        pltpu.VMEM((B_PER_W,), jnp.int32),
        pltpu.VMEM((CHUNK, EMB), jnp.float32),
        pltpu.VMEM((CHUNK, EMB), jnp.float32),
        pltpu.SemaphoreType.DMA,
        pltpu.SemaphoreType.DMA,
        pltpu.SemaphoreType.DMA,
        pltpu.SemaphoreType.DMA,
    ],
    compiler_params=pltpu.CompilerParams(use_tc_tiling_on_sc=False),
)


def kernel(x, table):
    tab = jnp.pad(table, ((0, VOCAB_PAD - table.shape[0]), (0, 0)))
    out = _sc_call(x.reshape(-1), tab)
    return out.reshape(x.shape[0], x.shape[1], EMB)
